# Initial kernel scaffold; baseline (speedup 1.0000x reference)
import jax
import jax.numpy as jnp
from jax.experimental import pallas as pl


def kernel(x, W1, as1, an1, b1, W2, as2, an2, b2, W3, as3, an3, b3, Wf1, bf1, Wf2, bf2, a):
    raise NotImplementedError("write your pallas kernel here")


# speedup vs baseline: 1.7131x; 1.7131x over previous
import jax
import jax.numpy as jnp
from jax import lax
from jax.experimental import pallas as pl
from jax.experimental.pallas import tpu as pltpu

N = 2048
F = 256
H = 3
C = 32
BM = 256
NB = N // BM


def _gat_kernel(x_ref, a_ref,
                W1_ref, as1_ref, an1_ref, b1_ref,
                W2_ref, as2_ref, an2_ref, b2_ref,
                W3_ref, as3_ref, an3_ref, b3_ref,
                Wf1_ref, bf1_ref, Wf2_ref, bf2_ref,
                out_ref,
                h_s, fs_s, fnT_s, buf0, buf1, pmax):
    l = pl.program_id(0)
    b = pl.program_id(1)

    def compute_h(inp, W_ref, as_ref, an_ref):
        h = jnp.dot(inp, W_ref[...], preferred_element_type=jnp.float32)
        h_s[...] = h
        for k in range(H):
            hk = h[:, C * k:C * (k + 1)]
            fs_s[:, k:k + 1] = lax.dot_general(
                hk, as_ref[k:k + 1, :], (((1,), (1,)), ((), ())),
                preferred_element_type=jnp.float32)
            fnT_s[k:k + 1, :] = lax.dot_general(
                an_ref[k:k + 1, :], hk, (((1,), (1,)), ((), ())),
                preferred_element_type=jnp.float32)

    @pl.when(jnp.logical_and(l == 0, b == 0))
    def _():
        compute_h(x_ref[...], W1_ref, as1_ref, an1_ref)

    @pl.when(jnp.logical_and(l == 1, b == 0))
    def _():
        compute_h(buf0[...], W2_ref, as2_ref, an2_ref)

    @pl.when(jnp.logical_and(l == 2, b == 0))
    def _():
        compute_h(buf1[...], W3_ref, as3_ref, an3_ref)

    mask = a_ref[...] != 0
    rows = pl.ds(b * BM, BM)
    acc = jnp.zeros((BM, C), jnp.float32)
    for k in range(H):
        z = fs_s[rows, k:k + 1] + fnT_s[k:k + 1, :]
        z = jnp.where(z >= 0, z, 0.2 * z)
        z = jnp.where(mask, z, -1e9)
        m = jnp.max(z, axis=1, keepdims=True)
        e = jnp.exp(z - m)
        den = jnp.sum(e, axis=1, keepdims=True)
        num = jnp.dot(e, h_s[:, C * k:C * (k + 1)],
                      preferred_element_type=jnp.float32)
        acc = acc + num / den

    @pl.when(l == 0)
    def _():
        buf0[rows, :] = jnp.maximum(acc * (1.0 / H) + b1_ref[...], 0.0)

    @pl.when(l == 1)
    def _():
        buf1[rows, :] = jnp.maximum(acc * (1.0 / H) + b2_ref[...], 0.0)

    @pl.when(l == 2)
    def _():
        xo = jnp.maximum(acc * (1.0 / H) + b3_ref[...], 0.0)
        bmax = jnp.max(xo, axis=0, keepdims=True)
        prev = jnp.where(b == 0, -jnp.inf, pmax[...])
        pmax[...] = jnp.maximum(prev, bmax)

    @pl.when(jnp.logical_and(l == 2, b == NB - 1))
    def _():
        p = pmax[...]
        hf = jnp.maximum(
            jnp.dot(p, Wf1_ref[...], preferred_element_type=jnp.float32)
            + bf1_ref[...], 0.0)
        out_ref[...] = (jnp.dot(hf, Wf2_ref[...],
                                preferred_element_type=jnp.float32)
                        + bf2_ref[...])


def kernel(x, W1, as1, an1, b1, W2, as2, an2, b2, W3, as3, an3, b3,
           Wf1, bf1, Wf2, bf2, a):
    a8 = a.astype(jnp.int8)

    def const(shape):
        return pl.BlockSpec(shape, lambda l, b: (0,) * len(shape))

    in_specs = [
        pl.BlockSpec((N, F), lambda l, b: (0, 0)),
        pl.BlockSpec((BM, N), lambda l, b: (b, 0)),
        const((F, H * C)), const((H, C)), const((H, C)), const((1, C)),
        const((C, H * C)), const((H, C)), const((H, C)), const((1, C)),
        const((C, H * C)), const((H, C)), const((H, C)), const((1, C)),
        const((C, 2 * C)), const((1, 2 * C)),
        const((2 * C, 1)), const((1, 1)),
    ]
    out = pl.pallas_call(
        _gat_kernel,
        grid=(3, NB),
        in_specs=in_specs,
        out_specs=pl.BlockSpec((1, 1), lambda l, b: (0, 0)),
        out_shape=jax.ShapeDtypeStruct((1, 1), jnp.float32),
        scratch_shapes=[
            pltpu.VMEM((N, H * C), jnp.float32),
            pltpu.VMEM((N, H), jnp.float32),
            pltpu.VMEM((8, N), jnp.float32),
            pltpu.VMEM((N, C), jnp.float32),
            pltpu.VMEM((N, C), jnp.float32),
            pltpu.VMEM((1, C), jnp.float32),
        ],
        compiler_params=pltpu.CompilerParams(
            dimension_semantics=("arbitrary", "arbitrary")),
    )(x, a8, W1, as1, an1, b1.reshape(1, C),
      W2, as2, an2, b2.reshape(1, C),
      W3, as3, an3, b3.reshape(1, C),
      Wf1, bf1.reshape(1, 2 * C), Wf2, bf2.reshape(1, 1))
    return out


# speedup vs baseline: 2.2508x; 1.3138x over previous
import jax
import jax.numpy as jnp
from jax import lax
from jax.experimental import pallas as pl
from jax.experimental.pallas import tpu as pltpu

N = 2048
F = 256
H = 3
C = 32
BM = 256
NB = N // BM


def _gat_kernel(x_ref, a_ref,
                W1_ref, as1_ref, an1_ref, b1_ref,
                W2_ref, as2_ref, an2_ref, b2_ref,
                W3_ref, as3_ref, an3_ref, b3_ref,
                Wf1_ref, bf1_ref, Wf2_ref, bf2_ref,
                out_ref,
                fs_s, fnT_s, rhs1_s, rhs2_s, aux_s, buf0, buf1, pmax):
    l = pl.program_id(0)
    b = pl.program_id(1)

    def compute_h(inp, W_ref, as_ref, an_ref):
        h = jnp.dot(inp, W_ref[...], preferred_element_type=jnp.float32)
        aux_s[0:1, 0:H * C] = jnp.mean(h, axis=0, keepdims=True)
        for k in range(H):
            hk = h[:, C * k:C * (k + 1)]
            fs_s[:, k:k + 1] = lax.dot_general(
                hk, as_ref[k:k + 1, :], (((1,), (1,)), ((), ())),
                preferred_element_type=jnp.float32)
            fn_col = lax.dot_general(
                hk, an_ref[k:k + 1, :], (((1,), (1,)), ((), ())),
                preferred_element_type=jnp.float32)
            fnT = lax.dot_general(
                an_ref[k:k + 1, :], hk, (((1,), (1,)), ((), ())),
                preferred_element_type=jnp.float32)
            fnT_s[k:k + 1, :] = fnT
            fnmax = jnp.max(fnT, axis=1, keepdims=True)
            aux_s[1:2, k:k + 1] = fnmax
            E1 = jnp.exp(fn_col - fnmax)
            E2 = jnp.exp(0.2 * (fn_col - fnmax))
            rhs1_s[:, 128 * k:128 * k + C] = hk * E1
            rhs1_s[:, 128 * k + C:128 * k + C + 1] = E1
            rhs2_s[:, 128 * k:128 * k + C] = hk * E2
            rhs2_s[:, 128 * k + C:128 * k + C + 1] = E2

    @pl.when(jnp.logical_and(l == 0, b == 0))
    def _():
        compute_h(x_ref[...], W1_ref, as1_ref, an1_ref)

    @pl.when(jnp.logical_and(l == 1, b == 0))
    def _():
        compute_h(buf0[...], W2_ref, as2_ref, an2_ref)

    @pl.when(jnp.logical_and(l == 2, b == 0))
    def _():
        compute_h(buf1[...], W3_ref, as3_ref, an3_ref)

    M_f = a_ref[...].astype(jnp.float32)
    rows = pl.ds(b * BM, BM)
    acc = jnp.zeros((BM, C), jnp.float32)
    for k in range(H):
        fs_blk = fs_s[rows, k:k + 1]
        fnmax = aux_s[1:2, k:k + 1]
        t = fs_blk + fnmax
        m = jnp.maximum(t, 0.2 * t)
        a1 = jnp.exp(t - m)
        a2 = jnp.exp(0.2 * t - m)
        s = fnT_s[k:k + 1, :] >= -fs_blk
        P1 = jnp.where(s, M_f, 0.0)
        P2 = M_f - P1
        Q1 = jnp.dot(P1, rhs1_s[:, 128 * k:128 * k + C + 1],
                     preferred_element_type=jnp.float32)
        Q2 = jnp.dot(P2, rhs2_s[:, 128 * k:128 * k + C + 1],
                     preferred_element_type=jnp.float32)
        num = a1 * Q1[:, :C] + a2 * Q2[:, :C]
        den = a1 * Q1[:, C:C + 1] + a2 * Q2[:, C:C + 1]
        r = jnp.where(den > 0, 1.0 / den, 0.0)
        acc = acc + jnp.where(den > 0, num * r,
                              aux_s[0:1, C * k:C * (k + 1)])

    @pl.when(l == 0)
    def _():
        buf0[rows, :] = jnp.maximum(acc * (1.0 / H) + b1_ref[...], 0.0)

    @pl.when(l == 1)
    def _():
        buf1[rows, :] = jnp.maximum(acc * (1.0 / H) + b2_ref[...], 0.0)

    @pl.when(l == 2)
    def _():
        xo = jnp.maximum(acc * (1.0 / H) + b3_ref[...], 0.0)
        bmax = jnp.max(xo, axis=0, keepdims=True)
        prev = jnp.where(b == 0, -jnp.inf, pmax[...])
        pmax[...] = jnp.maximum(prev, bmax)

    @pl.when(jnp.logical_and(l == 2, b == NB - 1))
    def _():
        p = pmax[...]
        hf = jnp.maximum(
            jnp.dot(p, Wf1_ref[...], preferred_element_type=jnp.float32)
            + bf1_ref[...], 0.0)
        out_ref[...] = (jnp.dot(hf, Wf2_ref[...],
                                preferred_element_type=jnp.float32)
                        + bf2_ref[...])


def kernel(x, W1, as1, an1, b1, W2, as2, an2, b2, W3, as3, an3, b3,
           Wf1, bf1, Wf2, bf2, a):
    a8 = a.astype(jnp.int8)

    def const(shape):
        return pl.BlockSpec(shape, lambda l, b: (0,) * len(shape))

    in_specs = [
        pl.BlockSpec((N, F), lambda l, b: (0, 0)),
        pl.BlockSpec((BM, N), lambda l, b: (b, 0)),
        const((F, H * C)), const((H, C)), const((H, C)), const((1, C)),
        const((C, H * C)), const((H, C)), const((H, C)), const((1, C)),
        const((C, H * C)), const((H, C)), const((H, C)), const((1, C)),
        const((C, 2 * C)), const((1, 2 * C)),
        const((2 * C, 1)), const((1, 1)),
    ]
    out = pl.pallas_call(
        _gat_kernel,
        grid=(3, NB),
        in_specs=in_specs,
        out_specs=pl.BlockSpec((1, 1), lambda l, b: (0, 0)),
        out_shape=jax.ShapeDtypeStruct((1, 1), jnp.float32),
        scratch_shapes=[
            pltpu.VMEM((N, H), jnp.float32),
            pltpu.VMEM((8, N), jnp.float32),
            pltpu.VMEM((N, 128 * H), jnp.float32),
            pltpu.VMEM((N, 128 * H), jnp.float32),
            pltpu.VMEM((8, 128), jnp.float32),
            pltpu.VMEM((N, C), jnp.float32),
            pltpu.VMEM((N, C), jnp.float32),
            pltpu.VMEM((1, C), jnp.float32),
        ],
        compiler_params=pltpu.CompilerParams(
            dimension_semantics=("arbitrary", "arbitrary")),
    )(x, a8, W1, as1, an1, b1.reshape(1, C),
      W2, as2, an2, b2.reshape(1, C),
      W3, as3, an3, b3.reshape(1, C),
      Wf1, bf1.reshape(1, 2 * C), Wf2, bf2.reshape(1, 1))
    return out
